# speedup vs baseline: 5.1604x; 5.1604x over previous
import functools

import jax
import jax.numpy as jnp
from jax import lax
from jax.experimental import pallas as pl
from jax.experimental.pallas import tpu as pltpu
from jax.experimental.pallas import tpu_sc as plsc

N_NODES = 10000
N_EDGES = 320000
FEAT = 128


_TC_BLK = 1000


def _tables_body(x_ref, wt_ref, b_ref, y1_ref, y2_ref):
    xb = x_ref[...]
    w1 = wt_ref[:FEAT, :]
    w2 = wt_ref[FEAT:, :]
    y1_ref[...] = jnp.dot(xb, w1, preferred_element_type=jnp.float32)
    y2_ref[...] = jnp.dot(xb, w2, preferred_element_type=jnp.float32) + b_ref[...]


def _node_tables(x, Wt, b2):
    n = x.shape[0]
    return pl.pallas_call(
        _tables_body,
        grid=(n // _TC_BLK,),
        in_specs=[
            pl.BlockSpec((_TC_BLK, FEAT), lambda i: (i, 0)),
            pl.BlockSpec((2 * FEAT, FEAT), lambda i: (0, 0)),
            pl.BlockSpec((1, FEAT), lambda i: (0, 0)),
        ],
        out_specs=[
            pl.BlockSpec((_TC_BLK, FEAT), lambda i: (i, 0)),
            pl.BlockSpec((_TC_BLK, FEAT), lambda i: (i, 0)),
        ],
        out_shape=[
            jax.ShapeDtypeStruct((n, FEAT), jnp.float32),
            jax.ShapeDtypeStruct((n, FEAT), jnp.float32),
        ],
    )(x, Wt, b2)


_NW = 32
_PER_W = N_EDGES // _NW
_C = 400
_NCHUNK = _PER_W // _C
_VECS = FEAT // 16


@functools.partial(
    pl.kernel,
    out_type=jax.ShapeDtypeStruct((N_EDGES, FEAT), jnp.float32),
    mesh=plsc.VectorSubcoreMesh(core_axis_name="c", subcore_axis_name="s"),
    scratch_types=[
        pltpu.VMEM((_C,), jnp.int32),
        pltpu.VMEM((_C,), jnp.int32),
        pltpu.VMEM((_C, FEAT), jnp.float32),
        pltpu.VMEM((_C, FEAT), jnp.float32),
        pltpu.SemaphoreType.DMA,
        pltpu.SemaphoreType.DMA,
    ],
)
def _sc_edges(y1_hbm, y2_hbm, src_hbm, dst_hbm, out_hbm,
              src_v, dst_v, rows_a, rows_b, sem_a, sem_b):
    wid = lax.axis_index("s") * 2 + lax.axis_index("c")
    base_w = wid * _PER_W

    def chunk_body(c, carry):
        base = base_w + c * _C
        pltpu.sync_copy(src_hbm.at[pl.ds(base, _C)], src_v)
        pltpu.sync_copy(dst_hbm.at[pl.ds(base, _C)], dst_v)
        cp_a = pltpu.async_copy(y1_hbm.at[src_v], rows_a, sem_a)
        cp_b = pltpu.async_copy(y2_hbm.at[dst_v], rows_b, sem_b)
        cp_a.wait()
        cp_b.wait()

        def row_body(e, carry2):
            for j in range(_VECS):
                sl = pl.ds(j * 16, 16)
                rows_a[e, sl] = jnp.maximum(rows_a[e, sl] + rows_b[e, sl], 0.0)
            return carry2

        lax.fori_loop(0, _C, row_body, 0, unroll=False)
        pltpu.sync_copy(rows_a, out_hbm.at[pl.ds(base, _C)])
        return carry

    lax.fori_loop(0, _NCHUNK, chunk_body, 0, unroll=False)


def kernel(x, edge_index, W, b):
    src = edge_index[0].astype(jnp.int32)
    dst = edge_index[1].astype(jnp.int32)
    y1, y2 = _node_tables(x, W.T, b.reshape(1, FEAT))
    return _sc_edges(y1, y2, src, dst)


# speedup vs baseline: 7.9188x; 1.5346x over previous
import functools

import jax
import jax.numpy as jnp
from jax import lax
from jax.experimental import pallas as pl
from jax.experimental.pallas import tpu as pltpu
from jax.experimental.pallas import tpu_sc as plsc

N_NODES = 10000
N_EDGES = 320000
FEAT = 128


_TC_BLK = 1000


def _tables_body(x_ref, wt_ref, b_ref, y1_ref, y2_ref):
    xb = x_ref[...]
    w1 = wt_ref[:FEAT, :]
    w2 = wt_ref[FEAT:, :]
    y1_ref[...] = jnp.dot(xb, w1, preferred_element_type=jnp.float32)
    y2_ref[...] = jnp.dot(xb, w2, preferred_element_type=jnp.float32) + b_ref[...]


def _node_tables(x, Wt, b2):
    n = x.shape[0]
    return pl.pallas_call(
        _tables_body,
        grid=(n // _TC_BLK,),
        in_specs=[
            pl.BlockSpec((_TC_BLK, FEAT), lambda i: (i, 0)),
            pl.BlockSpec((2 * FEAT, FEAT), lambda i: (0, 0)),
            pl.BlockSpec((1, FEAT), lambda i: (0, 0)),
        ],
        out_specs=[
            pl.BlockSpec((_TC_BLK, FEAT), lambda i: (i, 0)),
            pl.BlockSpec((_TC_BLK, FEAT), lambda i: (i, 0)),
        ],
        out_shape=[
            jax.ShapeDtypeStruct((n, FEAT), jnp.float32),
            jax.ShapeDtypeStruct((n, FEAT), jnp.float32),
        ],
    )(x, Wt, b2)


_NW = 32
_PER_W = N_EDGES // _NW
_C = 200
_NCHUNK = _PER_W // _C
_VECS = FEAT // 16


@functools.partial(
    pl.kernel,
    out_type=jax.ShapeDtypeStruct((N_EDGES, FEAT), jnp.float32),
    mesh=plsc.VectorSubcoreMesh(core_axis_name="c", subcore_axis_name="s"),
    scratch_types=[
        pltpu.VMEM((_PER_W,), jnp.int32),
        pltpu.VMEM((_PER_W,), jnp.int32),
        pltpu.VMEM((_C, FEAT), jnp.float32),
        pltpu.VMEM((_C, FEAT), jnp.float32),
        pltpu.VMEM((_C, FEAT), jnp.float32),
        pltpu.VMEM((_C, FEAT), jnp.float32),
        pltpu.SemaphoreType.DMA,
        pltpu.SemaphoreType.DMA,
        pltpu.SemaphoreType.DMA,
        pltpu.SemaphoreType.DMA,
    ],
)
def _sc_edges(y1_hbm, y2_hbm, src_hbm, dst_hbm, out_hbm,
              src_all, dst_all, rows_a0, rows_b0, rows_a1, rows_b1,
              sem_g0, sem_g1, sem_o0, sem_o1):
    wid = lax.axis_index("s") * 2 + lax.axis_index("c")
    base_w = wid * _PER_W

    rows_a = (rows_a0, rows_a1)
    rows_b = (rows_b0, rows_b1)
    sem_g = (sem_g0, sem_g1)
    sem_o = (sem_o0, sem_o1)

    pltpu.sync_copy(src_hbm.at[pl.ds(base_w, _PER_W)], src_all)
    pltpu.sync_copy(dst_hbm.at[pl.ds(base_w, _PER_W)], dst_all)

    def gather_start(c, k):
        off = c * _C
        pltpu.async_copy(y1_hbm.at[src_all.at[pl.ds(off, _C)]], rows_a[k], sem_g[k])
        pltpu.async_copy(y2_hbm.at[dst_all.at[pl.ds(off, _C)]], rows_b[k], sem_g[k])

    def gather_wait(k):
        pltpu.make_async_copy(y1_hbm.at[pl.ds(0, _C)], rows_a[k], sem_g[k]).wait()
        pltpu.make_async_copy(y2_hbm.at[pl.ds(0, _C)], rows_b[k], sem_g[k]).wait()

    def out_wait(k):
        pltpu.make_async_copy(rows_a[k], out_hbm.at[pl.ds(0, _C)], sem_o[k]).wait()

    def compute(k):
        def row_body(e, carry2):
            for j in range(_VECS):
                sl = pl.ds(j * 16, 16)
                rows_a[k][e, sl] = jnp.maximum(
                    rows_a[k][e, sl] + rows_b[k][e, sl], 0.0)
            return carry2

        lax.fori_loop(0, _C, row_body, 0, unroll=False)

    gather_start(0, 0)

    def pair_body(g, carry):
        for k in range(2):
            c = 2 * g + k

            @pl.when(c >= 1)
            def _():
                out_wait(1 - k)

            @pl.when(c < _NCHUNK - 1)
            def _():
                gather_start(c + 1, 1 - k)

            gather_wait(k)
            compute(k)
            pltpu.async_copy(rows_a[k], out_hbm.at[pl.ds(base_w + c * _C, _C)],
                             sem_o[k])
        return carry

    lax.fori_loop(0, _NCHUNK // 2, pair_body, 0, unroll=False)
    out_wait(1)


def kernel(x, edge_index, W, b):
    src = edge_index[0].astype(jnp.int32)
    dst = edge_index[1].astype(jnp.int32)
    y1, y2 = _node_tables(x, W.T, b.reshape(1, FEAT))
    return _sc_edges(y1, y2, src, dst)
